# baseline (speedup 1.0000x reference)
import functools

import jax
import jax.numpy as jnp
from jax import lax
from jax.experimental import pallas as pl
from jax.experimental.pallas import tpu as pltpu
from jax.experimental.pallas import tpu_sc as plsc

_L = 16


def _build_sc_kernel(B, T, E, scale):
    info = plsc.get_sparse_core_info()
    NC, NS = info.num_cores, info.num_subcores
    NW = NC * NS
    b_per_w = B // NW
    R = b_per_w * T
    BPC = 2
    CH = BPC * T
    NCHUNK = R // CH
    NVEC = CH // _L
    NSUB = 10
    SUBR = CH // NSUB
    mesh = plsc.VectorSubcoreMesh(core_axis_name="c", subcore_axis_name="s")

    @functools.partial(
        pl.kernel,
        out_type=jax.ShapeDtypeStruct((B * T, E), jnp.float32),
        mesh=mesh,
        compiler_params=pltpu.CompilerParams(use_tc_tiling_on_sc=False),
        scratch_types=[
            pltpu.VMEM((T, E), jnp.float32),
            pltpu.VMEM((R,), jnp.int32),
            pltpu.VMEM((R + _L,), jnp.float32),
            pltpu.VMEM((NCHUNK, CH), jnp.int32),
            pltpu.VMEM((2, CH, E), jnp.float32),
            pltpu.SemaphoreType.DMA,
            pltpu.SemaphoreType.DMA,
            pltpu.SemaphoreType.DMA,
            pltpu.SemaphoreType.DMA,
        ],
    )
    def sc_kernel(ids_hbm, maskf_hbm, table_hbm, pos_hbm, out_hbm,
                  pos_v, ids_v, mf_v, mid_v, rows_v,
                  gsem0, gsem1, wsem0, wsem1):
        gsem = (gsem0, gsem1)
        wsem = (wsem0, wsem1)
        wid = lax.axis_index("s") * NC + lax.axis_index("c")
        w0 = wid * R
        pltpu.sync_copy(pos_hbm, pos_v)
        pltpu.sync_copy(ids_hbm.at[pl.ds(w0, R)], ids_v)
        pltpu.sync_copy(maskf_hbm.at[pl.ds(w0, R)], mf_v.at[pl.ds(0, R)])

        def idx_chunk(g, carry):
            def idx_vec(v, c):
                off = g * CH + v * _L
                keep = mf_v[pl.ds(off, _L)] > 0.5
                mid_v[g, pl.ds(v * _L, _L)] = jnp.where(
                    keep, ids_v[pl.ds(off, _L)], 0)
                return c
            return lax.fori_loop(0, NVEC, idx_vec, carry)
        lax.fori_loop(0, NCHUNK, idx_chunk, 0)

        def gather(g, p):
            return [
                pltpu.async_copy(
                    table_hbm.at[mid_v.at[g].at[pl.ds(s * SUBR, SUBR)]],
                    rows_v.at[p].at[pl.ds(s * SUBR, SUBR)],
                    gsem[p])
                for s in range(NSUB)
            ]

        def compute(g, p):
            for sub in range(BPC):
                def body(t, c, sub=sub):
                    r = sub * T + t
                    m = mf_v[pl.ds(g * CH + r, _L)][0]
                    for q in range(E // _L):
                        sl = pl.ds(q * _L, _L)
                        rows_v[p, r, sl] = (rows_v[p, r, sl] * scale
                                            + pos_v[t, sl] * m)
                    return c
                lax.fori_loop(0, T, body, 0)

        def writeback(g, p):
            return pltpu.async_copy(
                rows_v.at[p], out_hbm.at[pl.ds(w0 + g * CH, CH)], wsem[p])

        cps = [None, None]
        wps = [None, None]
        cps[0] = gather(0, 0)
        cps[1] = gather(1, 1)
        for g in range(NCHUNK):
            p = g & 1
            for h in cps[p]:
                h.wait()
            compute(g, p)
            wps[p] = writeback(g, p)
            if g + 2 < NCHUNK:
                wps[p].wait()
                cps[p] = gather(g + 2, p)
        wps[0].wait()
        wps[1].wait()

    return sc_kernel


def kernel(item_id, padding_mask, item_table, pos_table):
    B, T = item_id.shape
    V, E = item_table.shape
    scale = float(E) ** 0.5
    ids = item_id.astype(jnp.int32).reshape(-1)
    maskf = padding_mask.astype(jnp.float32).reshape(-1)
    sc = _build_sc_kernel(B, T, E, scale)
    out = sc(ids, maskf, item_table, pos_table)
    return out.reshape(B, T, E)


# speedup vs baseline: 1.0042x; 1.0042x over previous
import functools

import jax
import jax.numpy as jnp
from jax import lax
from jax.experimental import pallas as pl
from jax.experimental.pallas import tpu as pltpu
from jax.experimental.pallas import tpu_sc as plsc

_L = 16


def _build_sc_kernel(B, T, E, scale):
    info = plsc.get_sparse_core_info()
    NC, NS = info.num_cores, info.num_subcores
    NW = NC * NS
    b_per_w = B // NW
    R = b_per_w * T
    BPC = 2
    CH = BPC * T
    NCHUNK = R // CH
    NVEC = CH // _L
    NSUB = 10
    SUBR = CH // NSUB
    mesh = plsc.VectorSubcoreMesh(core_axis_name="c", subcore_axis_name="s")

    @functools.partial(
        pl.kernel,
        out_type=jax.ShapeDtypeStruct((B * T, E), jnp.float32),
        mesh=mesh,
        compiler_params=pltpu.CompilerParams(use_tc_tiling_on_sc=False),
        scratch_types=[
            pltpu.VMEM((T, E), jnp.float32),
            pltpu.VMEM((R,), jnp.int32),
            pltpu.VMEM((R + _L,), jnp.float32),
            pltpu.VMEM((NCHUNK, CH), jnp.int32),
            pltpu.VMEM((2, CH, E), jnp.float32),
            pltpu.SemaphoreType.DMA,
            pltpu.SemaphoreType.DMA,
            pltpu.SemaphoreType.DMA,
            pltpu.SemaphoreType.DMA,
        ],
    )
    def sc_kernel(ids_hbm, maskf_hbm, table_hbm, pos_hbm, out_hbm,
                  pos_v, ids_v, mf_v, mid_v, rows_v,
                  gsem0, gsem1, wsem0, wsem1):
        gsem = (gsem0, gsem1)
        wsem = (wsem0, wsem1)
        wid = lax.axis_index("s") * NC + lax.axis_index("c")
        w0 = wid * R
        pltpu.sync_copy(pos_hbm, pos_v)
        pltpu.sync_copy(ids_hbm.at[pl.ds(w0, R)], ids_v)
        pltpu.sync_copy(maskf_hbm.at[pl.ds(w0, R)], mf_v.at[pl.ds(0, R)])

        def idx_chunk(g, carry):
            def idx_vec(v, c):
                off = g * CH + v * _L
                keep = mf_v[pl.ds(off, _L)] > 0.5
                mid_v[g, pl.ds(v * _L, _L)] = jnp.where(
                    keep, ids_v[pl.ds(off, _L)], 0)
                return c
            return lax.fori_loop(0, NVEC, idx_vec, carry)
        lax.fori_loop(0, NCHUNK, idx_chunk, 0)

        def gather(g, p):
            def issue(v, c):
                iv = mid_v[g, pl.ds(v * _L, _L)]
                pltpu.async_copy(
                    table_hbm.at[iv],
                    rows_v.at[p].at[pl.ds(v * _L, _L)],
                    gsem[p])
                return c
            lax.fori_loop(0, NVEC, issue, 0)

        def gather_wait(p):
            pltpu.make_async_copy(
                table_hbm.at[pl.ds(0, CH)], rows_v.at[p], gsem[p]).wait()

        def compute(g, p):
            for sub in range(BPC):
                def body(t, c, sub=sub):
                    r = sub * T + t
                    m = mf_v[pl.ds(g * CH + r, _L)][0]
                    for q in range(E // _L):
                        sl = pl.ds(q * _L, _L)
                        rows_v[p, r, sl] = (rows_v[p, r, sl] * scale
                                            + pos_v[t, sl] * m)
                    return c
                lax.fori_loop(0, T, body, 0)

        def writeback(g, p):
            return pltpu.async_copy(
                rows_v.at[p], out_hbm.at[pl.ds(w0 + g * CH, CH)], wsem[p])

        wps = [None, None]
        gather(0, 0)
        gather(1, 1)
        for g in range(NCHUNK):
            p = g & 1
            gather_wait(p)
            compute(g, p)
            wps[p] = writeback(g, p)
            if g + 2 < NCHUNK:
                wps[p].wait()
                gather(g + 2, p)
        wps[0].wait()
        wps[1].wait()

    return sc_kernel


def kernel(item_id, padding_mask, item_table, pos_table):
    B, T = item_id.shape
    V, E = item_table.shape
    scale = float(E) ** 0.5
    ids = item_id.astype(jnp.int32).reshape(-1)
    maskf = padding_mask.astype(jnp.float32).reshape(-1)
    sc = _build_sc_kernel(B, T, E, scale)
    out = sc(ids, maskf, item_table, pos_table)
    return out.reshape(B, T, E)
